# Initial kernel scaffold; baseline (speedup 1.0000x reference)
import jax
import jax.numpy as jnp
from jax.experimental import pallas as pl


def kernel(x, table):
    raise NotImplementedError("write your pallas kernel here")


# speedup vs baseline: 1.2738x; 1.2738x over previous
import functools

import jax
import jax.numpy as jnp
from jax import lax
from jax.experimental import pallas as pl
from jax.experimental.pallas import tpu as pltpu
from jax.experimental.pallas import tpu_sc as plsc

_D = 32
_B = 4096 * 200
_SCALE = float(_D) ** 0.5

_info = plsc.get_sparse_core_info()
_NC, _NS, _L = _info.num_cores, _info.num_subcores, _info.num_lanes
_NW = _NC * _NS

_GRP = 128
_GRP_PER_CHUNK = 8
_CHUNK = _GRP * _GRP_PER_CHUNK
_B_PER_W = _B // _NW
_N_CHUNKS = _B_PER_W // _CHUNK

_mesh = plsc.VectorSubcoreMesh(core_axis_name="c", subcore_axis_name="s")


@functools.partial(
    pl.kernel,
    mesh=_mesh,
    out_type=jax.ShapeDtypeStruct((_B, _D), jnp.float32),
    scratch_types=[
        pltpu.VMEM((_GRP_PER_CHUNK, _GRP), jnp.int32),
        pltpu.VMEM((_CHUNK, _D), jnp.float32),
        pltpu.SemaphoreType.DMA,
    ],
    compiler_params=pltpu.CompilerParams(use_tc_tiling_on_sc=False),
)
def _emb_lookup(table_hbm, idx_hbm, out_hbm, idx_v, rows_v, sem):
    wid = lax.axis_index("s") * _NC + lax.axis_index("c")
    base = wid * _B_PER_W

    def chunk_body(ci, carry):
        off = pl.multiple_of(base + ci * _CHUNK, _CHUNK)
        idx_row = pl.multiple_of(off // _GRP, _GRP_PER_CHUNK)
        pltpu.sync_copy(idx_hbm.at[pl.ds(idx_row, _GRP_PER_CHUNK)], idx_v)
        descs = [
            pltpu.async_copy(
                table_hbm.at[idx_v.at[j]],
                rows_v.at[pl.ds(j * _GRP, _GRP)],
                sem,
            )
            for j in range(_GRP_PER_CHUNK)
        ]
        for d in descs:
            d.wait()

        def scale_row(r, c):
            for h in range(2):
                sl = pl.ds(h * _L, _L)
                rows_v[r, sl] = rows_v[r, sl] * _SCALE
            return c

        lax.fori_loop(0, _CHUNK, scale_row, 0)

        pltpu.sync_copy(rows_v, out_hbm.at[pl.ds(off, _CHUNK)])
        return carry

    lax.fori_loop(0, _N_CHUNKS, chunk_body, 0)


def kernel(x, table):
    idx = x.reshape(_B // _GRP, _GRP).astype(jnp.int32)
    out = _emb_lookup(table, idx)
    return out.reshape(x.shape[0], x.shape[1], _D)
